# Initial kernel scaffold; baseline (speedup 1.0000x reference)
import jax
import jax.numpy as jnp
from jax.experimental import pallas as pl


def kernel(dgm, issublevel):
    raise NotImplementedError("write your pallas kernel here")


# speedup vs baseline: 3.2074x; 3.2074x over previous
import functools

import jax
import jax.numpy as jnp
import numpy as np
from jax import lax
from jax.experimental import pallas as pl
from jax.experimental.pallas import tpu as pltpu
from jax.experimental.pallas import tpu_sc as plsc

N = 1_000_000
K = 1024
NW = 32
NP = 1_000_448
SH = NP // NW
HALF = SH // 2
CHH = HALF // 16
CH = SH // 16
NB = 256
HSZ = NB * 16

_TOP = np.uint32(0x80000000)


def _worker_id():
    return lax.axis_index("s") * 2 + lax.axis_index("c")


def _merge_hist(cid, sid, hist_v, acc_v, tmp_v, shared, hist_hbm):
    pltpu.sync_copy(hist_v, shared.at[sid])
    plsc.subcore_barrier()
    colbase = sid * (HSZ // 16)

    def rbody(r, _):
        pltpu.sync_copy(shared.at[r, pl.ds(colbase, HSZ // 16)], tmp_v)

        def abody(j, _):
            plsc.addupdate(acc_v.at[pl.ds(j * 16, 16)], tmp_v[pl.ds(j * 16, 16)])
            return 0

        lax.fori_loop(0, HSZ // 256, abody, 0)
        return 0

    pltpu.sync_copy(shared.at[0, pl.ds(colbase, HSZ // 16)], acc_v)
    lax.fori_loop(1, 16, rbody, 0)
    pltpu.sync_copy(acc_v, hist_hbm.at[cid, pl.ds(colbase, HSZ // 16)])


@functools.cache
def _build():
    mesh = plsc.VectorSubcoreMesh(
        core_axis_name="c", subcore_axis_name="s", num_cores=2, num_subcores=16)

    @functools.partial(
        pl.kernel,
        compiler_params=pltpu.CompilerParams(needs_layout_passes=False),
        out_type=[
            jax.ShapeDtypeStruct((NP,), jnp.int32),
            jax.ShapeDtypeStruct((2, HSZ), jnp.int32),
        ],
        mesh=mesh,
        scratch_types=[
            pltpu.VMEM((HALF,), jnp.float32),
            pltpu.VMEM((HALF,), jnp.float32),
            pltpu.VMEM((HALF,), jnp.int32),
            pltpu.VMEM((HSZ,), jnp.int32),
            pltpu.VMEM((HSZ // 16,), jnp.int32),
            pltpu.VMEM((HSZ // 16,), jnp.int32),
            pltpu.VMEM((16,), jnp.float32),
            pltpu.VMEM_SHARED((16, HSZ), jnp.int32),
        ],
    )
    def _k1(d0_hbm, d1_hbm, sgn_hbm, keys_hbm, hist_hbm,
            d0_v, d1_v, keys_v, hist_v, acc_v, tmp_v, sgn_v, shared):
        cid = lax.axis_index("c")
        sid = lax.axis_index("s")
        base = _worker_id() * SH
        lane = lax.iota(jnp.int32, 16)
        ones = jnp.ones((16,), jnp.int32)
        nvec = jnp.full((16,), N, jnp.int32)

        pltpu.sync_copy(sgn_hbm, sgn_v)
        sgn = sgn_v[...]

        def zbody(j, _):
            hist_v[pl.ds(j * 16, 16)] = jnp.zeros((16,), jnp.int32)
            return 0

        lax.fori_loop(0, HSZ // 16, zbody, 0)

        for h in range(2):
            pltpu.sync_copy(d0_hbm.at[pl.ds(base + h * HALF, HALF)], d0_v)
            pltpu.sync_copy(d1_hbm.at[pl.ds(base + h * HALF, HALF)], d1_v)

            def body(i, _):
                d0c = d0_v[pl.ds(i * 16, 16)]
                d1c = d1_v[pl.ds(i * 16, 16)]
                l = (d1c - d0c) * sgn
                l = jnp.where(l == jnp.float32(jnp.inf), jnp.float32(0), l)
                l = jnp.where(l != l, jnp.float32(0), l)
                u = plsc.bitcast(l, jnp.uint32)
                key = jnp.where(u >= _TOP, ~u, u | _TOP)
                gidx = base + h * HALF + i * 16 + lane
                ok = gidx < nvec
                key = jnp.where(ok, key, jnp.uint32(0))
                keys_v[pl.ds(i * 16, 16)] = plsc.bitcast(key, jnp.int32)
                slot = (key >> jnp.uint32(24)).astype(jnp.int32) * 16 + lane
                plsc.addupdate_scatter(hist_v, [slot], ones, mask=ok)
                return 0

            lax.fori_loop(0, CHH, body, 0)
            pltpu.sync_copy(keys_v, keys_hbm.at[pl.ds(base + h * HALF, HALF)])

        _merge_hist(cid, sid, hist_v, acc_v, tmp_v, shared, hist_hbm)

    def _make_hist_pass(shift_prev, shift, dmask):
        @functools.partial(
            pl.kernel,
            compiler_params=pltpu.CompilerParams(needs_layout_passes=False),
            out_type=jax.ShapeDtypeStruct((2, HSZ), jnp.int32),
            mesh=mesh,
            scratch_types=[
                pltpu.VMEM((SH,), jnp.int32),
                pltpu.VMEM((HSZ,), jnp.int32),
                pltpu.VMEM((HSZ // 16,), jnp.int32),
                pltpu.VMEM((HSZ // 16,), jnp.int32),
                pltpu.VMEM((16,), jnp.uint32),
                pltpu.VMEM_SHARED((16, HSZ), jnp.int32),
            ],
        )
        def _hk(keys_hbm, pfx_hbm, hist_hbm,
                keys_v, hist_v, acc_v, tmp_v, pfx_v, shared):
            cid = lax.axis_index("c")
            sid = lax.axis_index("s")
            base = _worker_id() * SH
            lane = lax.iota(jnp.int32, 16)
            ones = jnp.ones((16,), jnp.int32)
            nvec = jnp.full((16,), N, jnp.int32)

            pltpu.sync_copy(keys_hbm.at[pl.ds(base, SH)], keys_v)
            pltpu.sync_copy(pfx_hbm, pfx_v)
            pfx = pfx_v[...]

            def zbody(j, _):
                hist_v[pl.ds(j * 16, 16)] = jnp.zeros((16,), jnp.int32)
                return 0

            lax.fori_loop(0, HSZ // 16, zbody, 0)

            def body(i, _):
                k = plsc.bitcast(keys_v[pl.ds(i * 16, 16)], jnp.uint32)
                gidx = base + i * 16 + lane
                m = ((k >> jnp.uint32(shift_prev)) == pfx) & (gidx < nvec)
                digit = ((k >> jnp.uint32(shift)) & jnp.uint32(dmask)).astype(
                    jnp.int32)
                slot = digit * 16 + lane
                plsc.addupdate_scatter(hist_v, [slot], ones, mask=m)
                return 0

            lax.fori_loop(0, CH, body, 0)
            _merge_hist(cid, sid, hist_v, acc_v, tmp_v, shared, hist_hbm)

        return _hk

    _k2 = _make_hist_pass(24, 16, 0xFF)
    _k3 = _make_hist_pass(16, 8, 0xFF)
    _k3b = _make_hist_pass(8, 0, 0xFF)

    @functools.partial(
        pl.kernel,
        compiler_params=pltpu.CompilerParams(needs_layout_passes=False),
        out_type=[
            jax.ShapeDtypeStruct((NW, 1040), jnp.int32),
            jax.ShapeDtypeStruct((NW * 16,), jnp.int32),
        ],
        mesh=mesh,
        scratch_types=[
            pltpu.VMEM((SH,), jnp.int32),
            pltpu.VMEM((1040,), jnp.int32),
            pltpu.VMEM((16,), jnp.int32),
            pltpu.VMEM((16,), jnp.uint32),
        ],
    )
    def _k4(keys_hbm, t_hbm, ck_hbm, cnt_hbm, keys_v, cbuf_v, cnt_v, t_v):
        base = _worker_id() * SH

        pltpu.sync_copy(keys_hbm.at[pl.ds(base, SH)], keys_v)
        pltpu.sync_copy(t_hbm, t_v)
        tval = t_v[...]

        def body(i, off):
            k = plsc.bitcast(keys_v[pl.ds(i * 16, 16)], jnp.uint32)
            m = k > tval
            cum = plsc.cumsum(m.astype(jnp.int32))
            idx = off + cum - 1
            plsc.store_scatter(cbuf_v, [idx], plsc.bitcast(k, jnp.int32), mask=m)
            return off + jnp.max(cum)

        off = lax.fori_loop(0, CH, body, jnp.int32(0))
        cnt_v[...] = jnp.broadcast_to(off, (16,))
        pltpu.sync_copy(cbuf_v, ck_hbm.at[_worker_id()])
        pltpu.sync_copy(cnt_v, cnt_hbm.at[pl.ds(_worker_id() * 16, 16)])

    @functools.partial(
        pl.kernel,
        compiler_params=pltpu.CompilerParams(needs_layout_passes=False),
        out_type=jax.ShapeDtypeStruct((2 * K,), jnp.float32),
        mesh=mesh,
        scratch_types=[
            pltpu.VMEM((8, 128), jnp.int32),
            pltpu.VMEM((K,), jnp.int32),
            pltpu.VMEM((16,), jnp.int32),
            pltpu.VMEM((32,), jnp.int32),
            pltpu.VMEM((32,), jnp.float32),
            pltpu.SemaphoreType.DMA,
        ],
    )
    def _k5(ck_hbm, sidx_hbm, cgt_hbm, out_hbm,
            sidx_v, keys_v, cgt_v, pos_v, val_v, sem):
        wid = _worker_id()
        lane = lax.iota(jnp.int32, 16)

        pltpu.sync_copy(sidx_hbm, sidx_v)
        pltpu.sync_copy(cgt_hbm, cgt_v)
        for c in range(8):
            pltpu.async_copy(ck_hbm.at[sidx_v.at[c]],
                             keys_v.at[pl.ds(c * 128, 128)], sem).wait()
        cgt = cgt_v[...]

        for j in range(2):
            mstart = wid * 32 + j * 16
            mkv = plsc.bitcast(keys_v[pl.ds(mstart, 16)], jnp.uint32)
            posv = jnp.zeros((16,), jnp.int32)
            for t in range(16):
                bc = jnp.broadcast_to(
                    jnp.sum(jnp.where(lane == t, mkv, jnp.uint32(0))), (16,))
                gsel = mstart + t

                def body(cc, acc):
                    kv = plsc.bitcast(keys_v[pl.ds(cc * 16, 16)], jnp.uint32)
                    pos = cc * 16 + lane
                    gt = (kv > bc) & (pos < cgt)
                    eq = (kv == bc) & (pos < jnp.minimum(gsel, cgt))
                    return (acc + plsc.all_reduce_population_count(gt)
                            + plsc.all_reduce_population_count(eq))

                rank = lax.fori_loop(0, K // 16, body,
                                     jnp.zeros((16,), jnp.int32))
                posv = jnp.where(lane == t, rank, posv)
            gvec = mstart + lane
            valid = gvec < cgt
            fpos = jnp.where(valid, posv, K + gvec)
            negk = mkv < _TOP
            bits = jnp.where(negk, ~mkv, mkv ^ _TOP)
            vals = plsc.bitcast(bits, jnp.float32)
            pos_v[pl.ds(j * 16, 16)] = fpos
            val_v[pl.ds(j * 16, 16)] = vals

        pltpu.async_copy(val_v, out_hbm.at[pos_v], sem).wait()

    return _k1, _k2, _k3, _k3b, _k4, _k5


def _select_bin(hist, k_rem, nb):
    suf = jnp.cumsum(hist[::-1])[::-1]
    b = jnp.max(jnp.where(suf >= k_rem, jnp.arange(nb, dtype=jnp.int32), -1))
    higher = suf[b] - hist[b]
    return b, (k_rem - higher).astype(jnp.int32)


def _lane_sum(h2):
    return h2.sum(axis=0).reshape(NB, 16).sum(axis=1)


def kernel(dgm, issublevel):
    _k1, _k2, _k3, _k3b, _k4, _k5 = _build()

    d0 = jnp.pad(dgm[1, 0], (0, NP - N))
    d1 = jnp.pad(dgm[1, 1], (0, NP - N))
    sgn = jnp.broadcast_to(
        jnp.where(issublevel, jnp.float32(1.0), jnp.float32(-1.0)), (16,))

    keys, h1 = _k1(d0, d1, sgn)
    b1, k1 = _select_bin(_lane_sum(h1), jnp.int32(K), NB)

    pfx1 = b1.astype(jnp.uint32)
    h2 = _k2(keys, jnp.broadcast_to(pfx1, (16,)))
    b2, k2 = _select_bin(_lane_sum(h2), k1, NB)

    pfx2 = (pfx1 << 8) | b2.astype(jnp.uint32)
    h3 = _k3(keys, jnp.broadcast_to(pfx2, (16,)))
    b3, k3 = _select_bin(_lane_sum(h3), k2, NB)

    pfx3 = (pfx2 << 8) | b3.astype(jnp.uint32)
    h4 = _k3b(keys, jnp.broadcast_to(pfx3, (16,)))
    b4, k4 = _select_bin(_lane_sum(h4), k3, NB)

    t_key = (pfx3 << 8) | b4.astype(jnp.uint32)
    cgt = jnp.int32(K) - k4

    ck, cnts = _k4(keys, jnp.broadcast_to(t_key, (16,)))
    counts = cnts[::16]
    offs = jnp.concatenate(
        [jnp.zeros((1,), counts.dtype), jnp.cumsum(counts)[:-1]])
    g = jnp.arange(K, dtype=jnp.int32)
    w = jnp.searchsorted(offs, g, side="right").astype(jnp.int32) - 1
    flat = w * 1040 + (g - offs[w])
    flat = jnp.where(g < cgt, flat, g).astype(jnp.int32)

    outb = _k5(ck.reshape(-1), flat.reshape(8, 128),
               jnp.broadcast_to(cgt, (16,)))

    fill_bits = jnp.where(t_key >= _TOP, t_key ^ _TOP, ~t_key)
    fill = lax.bitcast_convert_type(fill_bits, jnp.float32)
    return jnp.where(g < cgt, outb[:K], fill)


# speedup vs baseline: 6.4093x; 1.9983x over previous
import functools

import jax
import jax.numpy as jnp
import numpy as np
from jax import lax
from jax.experimental import pallas as pl
from jax.experimental.pallas import tpu as pltpu
from jax.experimental.pallas import tpu_sc as plsc

N = 1_000_000
K = 1024
NW = 32
NP = 1_000_448
SH = NP // NW
HALF = SH // 2
CHH = HALF // 16
CH = SH // 16
NB = 256
HSZ = NB * 16

_TOP = np.uint32(0x80000000)


def _zero_hist(hist_v):
    def zbody(j, _):
        hist_v[pl.ds(j * 16, 16)] = jnp.zeros((16,), jnp.int32)
        return 0

    lax.fori_loop(0, HSZ // 16, zbody, 0)


def _merge_binh(sid, hist_v, acc_v, tmp_v, btv_v, shared_h, shared_b, binh_v):
    pltpu.sync_copy(hist_v, shared_h.at[sid])
    plsc.subcore_barrier()
    colbase = sid * (HSZ // 16)
    pltpu.sync_copy(shared_h.at[0, pl.ds(colbase, HSZ // 16)], acc_v)

    def rbody(r, _):
        pltpu.sync_copy(shared_h.at[r, pl.ds(colbase, HSZ // 16)], tmp_v)

        def abody(j, _):
            plsc.addupdate(acc_v.at[pl.ds(j * 16, 16)], tmp_v[pl.ds(j * 16, 16)])
            return 0

        lax.fori_loop(0, HSZ // 256, abody, 0)
        return 0

    lax.fori_loop(1, 16, rbody, 0)
    lane = lax.iota(jnp.int32, 16)
    bt = jnp.zeros((16,), jnp.int32)
    for l in range(16):
        bt = jnp.where(lane == l, jnp.sum(acc_v[pl.ds(l * 16, 16)]), bt)
    btv_v[...] = bt
    pltpu.sync_copy(btv_v, shared_b.at[pl.ds(sid * 16, 16)])
    plsc.subcore_barrier()
    pltpu.sync_copy(shared_b, binh_v)


def _sc_select(binh_v, krem, ties=()):
    lane = lax.iota(jnp.int32, 16)

    def chunk(j):
        ch = binh_v[pl.ds(j * 16, 16)]
        idx = j * 16 + lane
        for d, c, a in ties:
            ch = ch + jnp.where(a & (idx == d), c, 0)
        return ch, idx

    bcnt = jnp.zeros((16,), jnp.int32)
    run = jnp.zeros((16,), jnp.int32)
    for j in range(15, -1, -1):
        ch, _ = chunk(j)
        pc = plsc.cumsum(ch)
        tot = jnp.sum(ch)
        suf = run + (tot - pc) + ch
        bcnt = bcnt + plsc.all_reduce_population_count(suf >= krem)
        run = run + tot
    b = bcnt - 1
    hi = jnp.zeros((16,), jnp.int32)
    for j in range(16):
        ch, idx = chunk(j)
        hi = hi + jnp.sum(jnp.where(idx > b, ch, 0))
    return b, krem - hi


@functools.cache
def _build():
    mesh = plsc.VectorSubcoreMesh(
        core_axis_name="c", subcore_axis_name="s", num_cores=2, num_subcores=16)

    @functools.partial(
        pl.kernel,
        compiler_params=pltpu.CompilerParams(needs_layout_passes=False),
        out_type=[
            jax.ShapeDtypeStruct((NW, 1040), jnp.int32),
            jax.ShapeDtypeStruct((NW * 16,), jnp.int32),
            jax.ShapeDtypeStruct((NW * 16,), jnp.int32),
            jax.ShapeDtypeStruct((NW * 16,), jnp.int32),
        ],
        mesh=mesh,
        scratch_types=[
            pltpu.VMEM((HALF,), jnp.float32),
            pltpu.VMEM((HALF,), jnp.float32),
            pltpu.VMEM((SH,), jnp.int32),
            pltpu.VMEM((HSZ,), jnp.int32),
            pltpu.VMEM((HSZ // 16,), jnp.int32),
            pltpu.VMEM((HSZ // 16,), jnp.int32),
            pltpu.VMEM((16,), jnp.int32),
            pltpu.VMEM((NB,), jnp.int32),
            pltpu.VMEM((16,), jnp.float32),
            pltpu.VMEM((1040,), jnp.int32),
            pltpu.VMEM((16,), jnp.int32),
            pltpu.VMEM_SHARED((16, HSZ), jnp.int32),
            pltpu.VMEM_SHARED((NB,), jnp.int32),
        ],
    )
    def _ka(d0_hbm, d1_hbm, sgn_hbm, ck_hbm, cnt_hbm, tk_hbm, k4_hbm,
            d0_v, d1_v, keys_v, hist_v, acc_v, tmp_v, btv_v, binh_v, sgn_v,
            cbuf_v, o16_v, shared_h, shared_b):
        cid = lax.axis_index("c")
        sid = lax.axis_index("s")
        wid = cid * 16 + sid
        base = wid * SH
        lane = lax.iota(jnp.int32, 16)
        ones = jnp.ones((16,), jnp.int32)
        nvec = jnp.full((16,), N, jnp.int32)

        pltpu.sync_copy(sgn_hbm, sgn_v)
        sgn = sgn_v[...]
        _zero_hist(hist_v)

        for h in range(2):
            pltpu.sync_copy(d0_hbm.at[pl.ds(base + h * HALF, HALF)], d0_v)
            pltpu.sync_copy(d1_hbm.at[pl.ds(base + h * HALF, HALF)], d1_v)

            def body(i, _):
                d0c = d0_v[pl.ds(i * 16, 16)]
                d1c = d1_v[pl.ds(i * 16, 16)]
                l = (d1c - d0c) * sgn
                l = jnp.where(l == jnp.float32(jnp.inf), jnp.float32(0), l)
                l = jnp.where(l != l, jnp.float32(0), l)
                u = plsc.bitcast(l, jnp.uint32)
                key = jnp.where(u >= _TOP, ~u, u | _TOP)
                gidx = base + h * HALF + i * 16 + lane
                ok = gidx < nvec
                key = jnp.where(ok, key, jnp.uint32(0))
                keys_v[pl.ds(h * HALF + i * 16, 16)] = plsc.bitcast(
                    key, jnp.int32)
                slot = (key >> jnp.uint32(24)).astype(jnp.int32) * 16 + lane
                plsc.addupdate_scatter(hist_v, [slot], ones, mask=ok)
                return 0

            lax.fori_loop(0, CHH, body, 0)

        _merge_binh(sid, hist_v, acc_v, tmp_v, btv_v, shared_h, shared_b,
                    binh_v)
        krem = jnp.full((16,), K, jnp.int32)
        b, krem = _sc_select(binh_v, krem)
        pfx = b.astype(jnp.uint32)

        for shift_prev, shift in ((24, 16), (16, 8), (8, 0)):
            _zero_hist(hist_v)

            def pbody(i, _):
                k = plsc.bitcast(keys_v[pl.ds(i * 16, 16)], jnp.uint32)
                m = (k >> jnp.uint32(shift_prev)) == pfx
                digit = ((k >> jnp.uint32(shift)) & jnp.uint32(0xFF)).astype(
                    jnp.int32)
                plsc.addupdate_scatter(hist_v, [digit * 16 + lane], ones,
                                       mask=m)
                return 0

            lax.fori_loop(0, CH, pbody, 0)
            _merge_binh(sid, hist_v, acc_v, tmp_v, btv_v, shared_h, shared_b,
                        binh_v)
            b, krem = _sc_select(binh_v, krem)
            pfx = (pfx << jnp.uint32(8)) | b.astype(jnp.uint32)

        tval = pfx

        def cbody(i, off):
            k = plsc.bitcast(keys_v[pl.ds(i * 16, 16)], jnp.uint32)
            m = k > tval
            cum = plsc.cumsum(m.astype(jnp.int32))
            idx = off + cum - 1
            plsc.store_scatter(cbuf_v, [idx], plsc.bitcast(k, jnp.int32),
                               mask=m)
            return off + jnp.max(cum)

        off = lax.fori_loop(0, CH, cbody, jnp.int32(0))
        pltpu.sync_copy(cbuf_v, ck_hbm.at[wid])
        o16_v[...] = jnp.broadcast_to(off, (16,))
        pltpu.sync_copy(o16_v, cnt_hbm.at[pl.ds(wid * 16, 16)])
        o16_v[...] = plsc.bitcast(tval, jnp.int32)
        pltpu.sync_copy(o16_v, tk_hbm.at[pl.ds(wid * 16, 16)])
        o16_v[...] = krem
        pltpu.sync_copy(o16_v, k4_hbm.at[pl.ds(wid * 16, 16)])

    SB = NW * 1040

    @functools.partial(
        pl.kernel,
        compiler_params=pltpu.CompilerParams(needs_layout_passes=False),
        out_type=[
            jax.ShapeDtypeStruct((2 * K,), jnp.float32),
            jax.ShapeDtypeStruct((NW * 16,), jnp.int32),
            jax.ShapeDtypeStruct((NW * 16,), jnp.int32),
            jax.ShapeDtypeStruct((SB,), jnp.int32),
        ],
        mesh=mesh,
        scratch_types=[
            pltpu.VMEM((1040,), jnp.int32),
            pltpu.VMEM((1040,), jnp.int32),
            pltpu.VMEM((32,), jnp.int32),
            pltpu.VMEM((64,), jnp.int32),
            pltpu.VMEM((HSZ,), jnp.int32),
            pltpu.VMEM((HSZ // 16,), jnp.int32),
            pltpu.VMEM((HSZ // 16,), jnp.int32),
            pltpu.VMEM((16,), jnp.int32),
            pltpu.VMEM((NB,), jnp.int32),
            pltpu.VMEM((1040,), jnp.int32),
            pltpu.VMEM((16,), jnp.int32),
            pltpu.VMEM((1040,), jnp.int32),
            pltpu.VMEM((1040,), jnp.int32),
            pltpu.VMEM((32,), jnp.int32),
            pltpu.VMEM((32,), jnp.float32),
            pltpu.VMEM_SHARED((16, HSZ), jnp.int32),
            pltpu.VMEM_SHARED((NB,), jnp.int32),
            pltpu.SemaphoreType.DMA,
        ],
    )
    def _kb(ck_hbm, cnt_hbm, meta_hbm, out_hbm, tg_hbm, cg_hbm, st_hbm,
            ck0_v, ck1_v, cnt_v, meta_v, hist_v, acc_v, tmp_v, btv_v, binh_v,
            se_v, o16_v, vb_v, pb_v, pos_v, val_v, shared_h, shared_b, sem):
        cid = lax.axis_index("c")
        sid = lax.axis_index("s")
        wid = cid * 16 + sid
        lane = lax.iota(jnp.int32, 16)
        ones = jnp.ones((16,), jnp.int32)

        pltpu.sync_copy(cnt_hbm, cnt_v)
        pltpu.sync_copy(meta_hbm, meta_v)
        t0 = plsc.bitcast(meta_v[pl.ds(0, 16)], jnp.uint32)
        t1 = plsc.bitcast(meta_v[pl.ds(16, 16)], jnp.uint32)
        k40 = meta_v[pl.ds(32, 16)]
        k41 = meta_v[pl.ds(48, 16)]
        cnt0 = jnp.sum(jnp.where(lane == sid, cnt_v[pl.ds(0, 16)], 0))
        cnt1 = jnp.sum(jnp.where(lane == sid, cnt_v[pl.ds(16, 16)], 0))

        pltpu.sync_copy(ck_hbm.at[pl.ds(sid * 1040, 1040)], ck0_v)
        pltpu.sync_copy(ck_hbm.at[pl.ds((16 + sid) * 1040, 1040)], ck1_v)

        krem = jnp.full((16,), K, jnp.int32)
        pfx = jnp.zeros((16,), jnp.uint32)
        for pi, (shift_prev, shift) in enumerate(
                ((32, 24), (24, 16), (16, 8), (8, 0))):
            _zero_hist(hist_v)

            for ckr_v, cnt_r in ((ck0_v, cnt0), (ck1_v, cnt1)):
                def hbody(i, _, ckr_v=ckr_v, cnt_r=cnt_r):
                    k = plsc.bitcast(ckr_v[pl.ds(i * 16, 16)], jnp.uint32)
                    m = (i * 16 + lane) < cnt_r
                    if pi:
                        m = m & ((k >> jnp.uint32(shift_prev)) == pfx)
                    digit = ((k >> jnp.uint32(shift))
                             & jnp.uint32(0xFF)).astype(jnp.int32)
                    plsc.addupdate_scatter(hist_v, [digit * 16 + lane], ones,
                                           mask=m)
                    return 0

                lax.fori_loop(0, 65, hbody, 0)

            _merge_binh(sid, hist_v, acc_v, tmp_v, btv_v, shared_h, shared_b,
                        binh_v)
            ties = []
            for tk, tc in ((t0, k40), (t1, k41)):
                d = ((tk >> jnp.uint32(shift)) & jnp.uint32(0xFF)).astype(
                    jnp.int32)
                if pi:
                    a = (tk >> jnp.uint32(shift_prev)) == pfx
                else:
                    a = jnp.full((16,), True)
                ties.append((d, tc, a))
            b, krem = _sc_select(binh_v, krem, ties)
            pfx = (pfx << jnp.uint32(8)) | b.astype(jnp.uint32)

        tg = pfx
        cgt = jnp.full((16,), K, jnp.int32) - krem

        off = jnp.zeros((16,), jnp.int32)
        for ckr_v, cnt_r in ((ck0_v, cnt0), (ck1_v, cnt1)):
            def wbody(i, off, ckr_v=ckr_v, cnt_r=cnt_r):
                k = plsc.bitcast(ckr_v[pl.ds(i * 16, 16)], jnp.uint32)
                m = ((i * 16 + lane) < cnt_r) & (k > tg)
                cum = plsc.cumsum(m.astype(jnp.int32))
                plsc.store_scatter(vb_v, [off + cum - 1],
                                   plsc.bitcast(k, jnp.int32), mask=m)
                return off + jnp.max(cum)

            off = lax.fori_loop(0, 65, wbody, off)

        pltpu.sync_copy(vb_v, st_hbm.at[pl.ds(wid * 1040, 1040)])
        o16_v[...] = off
        pltpu.sync_copy(o16_v, shared_b.at[pl.ds(sid * 16, 16)])
        plsc.subcore_barrier()
        pltpu.sync_copy(shared_b, binh_v)
        cv = jnp.zeros((16,), jnp.int32)
        for r in range(16):
            cv = jnp.where(lane == r, binh_v[pl.ds(r * 16, 16)], cv)
        mvex = plsc.cumsum(cv) - cv
        ms = [jnp.broadcast_to(jnp.sum(jnp.where(lane == r, mvex, 0)), (16,))
              for r in range(16)]

        def ibody(i, _):
            jv = i * 16 + lane
            rsel = jnp.zeros((16,), jnp.int32)
            bsel = jnp.zeros((16,), jnp.int32)
            for r in range(16):
                ge = jv >= ms[r]
                rsel = rsel + ge.astype(jnp.int32)
                bsel = jnp.where(ge, ms[r], bsel)
            pb_v[pl.ds(i * 16, 16)] = ((rsel - 1) * 1040 + (jv - bsel)
                                       + cid * 16640)
            return 0

        lax.fori_loop(0, 65, ibody, 0)
        pltpu.async_copy(st_hbm.at[pb_v], se_v, sem).wait()

        for j in range(2):
            mstart = cid * 512 + sid * 32 + j * 16
            mkv = plsc.bitcast(se_v[pl.ds(mstart, 16)], jnp.uint32)
            posv = jnp.zeros((16,), jnp.int32)
            for t in range(16):
                bc = jnp.broadcast_to(
                    jnp.sum(jnp.where(lane == t, mkv, jnp.uint32(0))), (16,))
                gsel = mstart + t

                def rbody(ci, acc):
                    kv = plsc.bitcast(se_v[pl.ds(ci * 16, 16)], jnp.uint32)
                    p = ci * 16 + lane
                    gt = (kv > bc) & (p < cgt)
                    eq = (kv == bc) & (p < jnp.minimum(gsel, cgt))
                    return (acc + plsc.all_reduce_population_count(gt)
                            + plsc.all_reduce_population_count(eq))

                rank = lax.fori_loop(0, K // 16, rbody,
                                     jnp.zeros((16,), jnp.int32))
                posv = jnp.where(lane == t, rank, posv)
            gvec = mstart + lane
            valid = gvec < cgt
            fpos = jnp.where(valid, posv, K + gvec)
            negk = mkv < _TOP
            bits = jnp.where(negk, ~mkv, mkv ^ _TOP)
            pos_v[pl.ds(j * 16, 16)] = fpos
            val_v[pl.ds(j * 16, 16)] = plsc.bitcast(bits, jnp.float32)

        pltpu.async_copy(val_v, out_hbm.at[pos_v], sem).wait()
        o16_v[...] = plsc.bitcast(tg, jnp.int32)
        pltpu.sync_copy(o16_v, tg_hbm.at[pl.ds(wid * 16, 16)])
        o16_v[...] = cgt
        pltpu.sync_copy(o16_v, cg_hbm.at[pl.ds(wid * 16, 16)])

    return _ka, _kb


def kernel(dgm, issublevel):
    _ka, _kb = _build()

    d0 = jnp.pad(dgm[1, 0], (0, NP - N))
    d1 = jnp.pad(dgm[1, 1], (0, NP - N))
    sgn = jnp.broadcast_to(
        jnp.where(issublevel, jnp.float32(1.0), jnp.float32(-1.0)), (16,))

    ck, cnts, tks, k4s = _ka(d0, d1, sgn)
    counts = cnts[::16]
    meta = jnp.concatenate([
        jnp.broadcast_to(tks[0], (16,)), jnp.broadcast_to(tks[256], (16,)),
        jnp.broadcast_to(k4s[0], (16,)), jnp.broadcast_to(k4s[256], (16,)),
    ]).astype(jnp.int32)

    outb, tg, cg, _ = _kb(ck.reshape(-1), counts, meta)
    cgt = cg[0]
    tg_key = lax.bitcast_convert_type(tg[0], jnp.uint32)
    fill_bits = jnp.where(tg_key >= _TOP, tg_key ^ _TOP, ~tg_key)
    fill = lax.bitcast_convert_type(fill_bits, jnp.float32)
    g = jnp.arange(K, dtype=jnp.int32)
    return jnp.where(g < cgt, outb[:K], fill)
